# Initial kernel scaffold; baseline (speedup 1.0000x reference)
import functools

import jax
import jax.numpy as jnp
from jax import lax
from jax.experimental import pallas as pl
from jax.experimental.pallas import tpu as pltpu
from jax.experimental.pallas import tpu_sc as plsc

_N = 10000
_E = 320000
_F = 128
_K = 5

_NC = 2
_NS = 16
_NW = _NC * _NS
_EPT = _E // _NW
_CH = 80
_NCHUNK = _EPT // _CH
_NPAD = 10240

_mesh = plsc.VectorSubcoreMesh(core_axis_name="c", subcore_axis_name="s")


@functools.partial(
    pl.kernel,
    out_type=jax.ShapeDtypeStruct((_NC, _NPAD, 16), jnp.float32),
    mesh=_mesh,
    scratch_types=[
        pltpu.VMEM((_NCHUNK, _CH), jnp.int32),
        pltpu.VMEM((_CH, 16), jnp.float32),
        pltpu.VMEM((_NPAD // _NS, 16), jnp.float32),
        pltpu.VMEM_SHARED((_NPAD, 16), jnp.float32),
    ],
)
def _deg_kernel(dst_hbm, out_hbm, dst_v, ones_v, stage_v, acc_sp):
    cid = lax.axis_index("c")
    sid = lax.axis_index("s")
    wid = cid * _NS + sid
    rows = _NPAD // _NS

    zeros16 = jnp.zeros((16,), jnp.float32)
    ones16 = jnp.ones((16,), jnp.float32)

    def zero_body(r, _):
        stage_v[r] = zeros16
        return 0

    lax.fori_loop(0, rows, zero_body, 0)
    pltpu.sync_copy(stage_v, acc_sp.at[pl.ds(sid * rows, rows)])

    def ones_body(r, _):
        ones_v[r] = ones16
        return 0

    lax.fori_loop(0, _CH, ones_body, 0)

    pltpu.sync_copy(dst_hbm.at[wid], dst_v)
    plsc.subcore_barrier()

    def edge_body(j, _):
        pltpu.sync_copy(ones_v, acc_sp.at[dst_v.at[j]], add=True)
        return 0

    lax.fori_loop(0, _NCHUNK, edge_body, 0)
    plsc.subcore_barrier()

    pltpu.sync_copy(acc_sp.at[pl.ds(sid * rows, rows)], stage_v)
    pltpu.sync_copy(stage_v, out_hbm.at[cid, pl.ds(sid * rows, rows)])


@functools.partial(
    pl.kernel,
    out_type=jax.ShapeDtypeStruct((_NC, _N, _F), jnp.float32),
    mesh=_mesh,
    scratch_types=[
        pltpu.VMEM((_NCHUNK, _CH), jnp.int32),
        pltpu.VMEM((_NCHUNK, _CH), jnp.int32),
        pltpu.VMEM((_CH, _F), jnp.float32),
        pltpu.VMEM((_CH, _F), jnp.float32),
        pltpu.VMEM((_N // _NS, _F), jnp.float32),
        pltpu.VMEM_SHARED((_N, _F), jnp.float32),
        pltpu.SemaphoreType.DMA,
        pltpu.SemaphoreType.DMA,
    ],
)
def _prop_kernel(g_hbm, src_hbm, dst_hbm, out_hbm,
                 src_v, dst_v, rows_a, rows_b, stage_v, acc_sp, sem_a, sem_b):
    cid = lax.axis_index("c")
    sid = lax.axis_index("s")
    wid = cid * _NS + sid
    rows = _N // _NS

    zeros16 = jnp.zeros((16,), jnp.float32)

    def zero_body(i, _):
        r = i >> 3
        c = i & 7
        stage_v[r, pl.ds(c * 16, 16)] = zeros16
        return 0

    lax.fori_loop(0, rows * (_F // 16), zero_body, 0)
    pltpu.sync_copy(stage_v, acc_sp.at[pl.ds(sid * rows, rows)])

    pltpu.sync_copy(src_hbm.at[wid], src_v)
    pltpu.sync_copy(dst_hbm.at[wid], dst_v)
    plsc.subcore_barrier()

    pltpu.async_copy(g_hbm.at[src_v.at[0]], rows_a, sem_a)

    def pair_body(t, _):
        ja = 2 * t
        jb = 2 * t + 1
        jn = 2 * t + 2
        pltpu.async_copy(g_hbm.at[src_v.at[jb]], rows_b, sem_b)
        pltpu.make_async_copy(g_hbm.at[src_v.at[ja]], rows_a, sem_a).wait()
        pltpu.sync_copy(rows_a, acc_sp.at[dst_v.at[ja]], add=True)
        pltpu.async_copy(g_hbm.at[src_v.at[jn]], rows_a, sem_a)
        pltpu.make_async_copy(g_hbm.at[src_v.at[jb]], rows_b, sem_b).wait()
        pltpu.sync_copy(rows_b, acc_sp.at[dst_v.at[jb]], add=True)
        return 0

    lax.fori_loop(0, (_NCHUNK - 1) // 2, pair_body, 0)
    last = _NCHUNK - 1
    pltpu.make_async_copy(g_hbm.at[src_v.at[last]], rows_a, sem_a).wait()
    pltpu.sync_copy(rows_a, acc_sp.at[dst_v.at[last]], add=True)

    plsc.subcore_barrier()
    pltpu.sync_copy(acc_sp.at[pl.ds(sid * rows, rows)], stage_v)
    pltpu.sync_copy(stage_v, out_hbm.at[cid, pl.ds(sid * rows, rows)])


def _prop(g, src_r, dst_r):
    return _prop_kernel(g, src_r, dst_r)


def _dinv_call(degp):
    def body(p_ref, d1_ref, d2_ref):
        deg = p_ref[0] + p_ref[1]
        d = jnp.where(deg > 0, lax.rsqrt(jnp.maximum(deg, 1.0)), 0.0)
        d1_ref[...] = d
        d2_ref[...] = d * d

    return pl.pallas_call(
        body,
        out_shape=(
            jax.ShapeDtypeStruct((_NPAD * 16 // 128, 128), jnp.float32),
            jax.ShapeDtypeStruct((_NPAD * 16 // 128, 128), jnp.float32),
        ),
    )(degp)


_BN = 2000


def _scale_call(x, d1):
    def body(x_ref, d_ref, o_ref):
        o_ref[...] = x_ref[...] * d_ref[...]

    return pl.pallas_call(
        body,
        grid=(_N // _BN,),
        in_specs=[
            pl.BlockSpec((_BN, _F), lambda i: (i, 0)),
            pl.BlockSpec((_BN, 1), lambda i: (i, 0)),
        ],
        out_specs=pl.BlockSpec((_BN, _F), lambda i: (i, 0)),
        out_shape=jax.ShapeDtypeStruct((_N, _F), jnp.float32),
    )(x, d1)


def _step0_call(s, d1, d2):
    def body(s_ref, d1_ref, d2_ref, tx_ref, g_ref):
        ssum = s_ref[0] + s_ref[1]
        tx_ref[...] = -d1_ref[...] * ssum
        g_ref[...] = -d2_ref[...] * ssum

    return pl.pallas_call(
        body,
        grid=(_N // _BN,),
        in_specs=[
            pl.BlockSpec((_NC, _BN, _F), lambda i: (0, i, 0)),
            pl.BlockSpec((_BN, 1), lambda i: (i, 0)),
            pl.BlockSpec((_BN, 1), lambda i: (i, 0)),
        ],
        out_specs=(
            pl.BlockSpec((_BN, _F), lambda i: (i, 0)),
            pl.BlockSpec((_BN, _F), lambda i: (i, 0)),
        ),
        out_shape=(
            jax.ShapeDtypeStruct((_N, _F), jnp.float32),
            jax.ShapeDtypeStruct((_N, _F), jnp.float32),
        ),
    )(s, d1, d2)


def _stepk_call(s, d1, d2, tx_prev, g_prev):
    def body(s_ref, d1_ref, d2_ref, tp_ref, gp_ref, tx_ref, g_ref):
        ssum = s_ref[0] + s_ref[1]
        tx_ref[...] = -2.0 * d1_ref[...] * ssum - tp_ref[...]
        g_ref[...] = -2.0 * d2_ref[...] * ssum - gp_ref[...]

    return pl.pallas_call(
        body,
        grid=(_N // _BN,),
        in_specs=[
            pl.BlockSpec((_NC, _BN, _F), lambda i: (0, i, 0)),
            pl.BlockSpec((_BN, 1), lambda i: (i, 0)),
            pl.BlockSpec((_BN, 1), lambda i: (i, 0)),
            pl.BlockSpec((_BN, _F), lambda i: (i, 0)),
            pl.BlockSpec((_BN, _F), lambda i: (i, 0)),
        ],
        out_specs=(
            pl.BlockSpec((_BN, _F), lambda i: (i, 0)),
            pl.BlockSpec((_BN, _F), lambda i: (i, 0)),
        ),
        out_shape=(
            jax.ShapeDtypeStruct((_N, _F), jnp.float32),
            jax.ShapeDtypeStruct((_N, _F), jnp.float32),
        ),
    )(s, d1, d2, tx_prev, g_prev)


_BM = 500


def _matmul_call(x, tx1, tx2, tx3, s3, d1, W, b):
    def body(x_ref, t1_ref, t2_ref, t3_ref, s3_ref, d_ref, w_ref, b_ref, o_ref):
        tx4 = -2.0 * d_ref[...] * (s3_ref[0] + s3_ref[1]) - t2_ref[...]
        acc = jnp.dot(x_ref[...], w_ref[0], preferred_element_type=jnp.float32)
        acc += jnp.dot(t1_ref[...], w_ref[1], preferred_element_type=jnp.float32)
        acc += jnp.dot(t2_ref[...], w_ref[2], preferred_element_type=jnp.float32)
        acc += jnp.dot(t3_ref[...], w_ref[3], preferred_element_type=jnp.float32)
        acc += jnp.dot(tx4, w_ref[4], preferred_element_type=jnp.float32)
        o_ref[...] = acc + b_ref[...]

    return pl.pallas_call(
        body,
        grid=(_N // _BM,),
        in_specs=[
            pl.BlockSpec((_BM, _F), lambda i: (i, 0)),
            pl.BlockSpec((_BM, _F), lambda i: (i, 0)),
            pl.BlockSpec((_BM, _F), lambda i: (i, 0)),
            pl.BlockSpec((_BM, _F), lambda i: (i, 0)),
            pl.BlockSpec((_NC, _BM, _F), lambda i: (0, i, 0)),
            pl.BlockSpec((_BM, 1), lambda i: (i, 0)),
            pl.BlockSpec((_K, _F, _F), lambda i: (0, 0, 0)),
            pl.BlockSpec((1, _F), lambda i: (0, 0)),
        ],
        out_specs=pl.BlockSpec((_BM, _F), lambda i: (i, 0)),
        out_shape=jax.ShapeDtypeStruct((_N, _F), jnp.float32),
    )(x, tx1, tx2, tx3, s3, d1, W, b)


def kernel(x, adj, W, b):
    assert x.shape == (_N, _F) and adj.shape == (2, _E) and W.shape[0] == _K
    adj = adj.astype(jnp.int32)
    src_r = adj[0].reshape(_NW, _NCHUNK, _CH)
    dst_r = adj[1].reshape(_NW, _NCHUNK, _CH)

    degp = _deg_kernel(dst_r)
    d1_full, d2_full = _dinv_call(degp.reshape(_NC, _NPAD * 16 // 128, 128))
    d1 = d1_full.reshape(_NPAD, 16)[:_N, 0:1]
    d2 = d2_full.reshape(_NPAD, 16)[:_N, 0:1]

    g0 = _scale_call(x, d1)
    s0 = _prop(g0, src_r, dst_r)
    tx1, g1 = _step0_call(s0, d1, d2)
    s1 = _prop(g1, src_r, dst_r)
    tx2, g2 = _stepk_call(s1, d1, d2, x, g0)
    s2 = _prop(g2, src_r, dst_r)
    tx3, g3 = _stepk_call(s2, d1, d2, tx1, g1)
    s3 = _prop(g3, src_r, dst_r)
    out = _matmul_call(x, tx1, tx2, tx3, s3, d1, W, b.reshape(1, _F))
    return out


# speedup vs baseline: 7.1031x; 7.1031x over previous
import functools

import jax
import jax.numpy as jnp
from jax import lax
from jax.experimental import pallas as pl
from jax.experimental.pallas import tpu as pltpu
from jax.experimental.pallas import tpu_sc as plsc

_N = 10000
_E = 320000
_F = 128
_K = 5

_NC = 2
_NS = 16
_NW = _NC * _NS
_EPT = _E // _NW
_CH = 80
_NCHUNK = _EPT // _CH
_NPAD = 10240
_DW = 32

_mesh = plsc.VectorSubcoreMesh(core_axis_name="c", subcore_axis_name="s")


@functools.partial(
    pl.kernel,
    out_type=jax.ShapeDtypeStruct((_NC, _NPAD, _F), jnp.float32),
    mesh=_mesh,
    scratch_types=[
        pltpu.VMEM((_CH,), jnp.int32),
        pltpu.VMEM((_CH, _F), jnp.float32),
        pltpu.VMEM_SHARED((_NPAD, _F), jnp.float32),
    ],
)
def _deg_kernel(dst_hbm, out_hbm, dch, ones_v, acc_sp):
    cid = lax.axis_index("c")
    sid = lax.axis_index("s")
    wid = cid * _NS + sid
    rows = _NPAD // _NS

    zeros16 = jnp.zeros((16,), jnp.float32)
    ones16 = jnp.ones((16,), jnp.float32)

    def zero_body(i, _):
        ones_v[i >> 3, pl.ds((i & 7) * 16, 16)] = zeros16
        return 0

    lax.fori_loop(0, _CH * (_F // 16), zero_body, 0)
    for t in range(rows // _CH):
        pltpu.sync_copy(ones_v, acc_sp.at[pl.ds(sid * rows + t * _CH, _CH)])

    def ones_body(i, _):
        ones_v[i >> 3, pl.ds((i & 7) * 16, 16)] = ones16
        return 0

    lax.fori_loop(0, _CH * (_F // 16), ones_body, 0)
    plsc.subcore_barrier()

    def edge_body(j, _):
        pltpu.sync_copy(dst_hbm.at[pl.ds(wid * _EPT + j * _CH, _CH)], dch)
        pltpu.sync_copy(ones_v, acc_sp.at[dch], add=True)
        return 0

    lax.fori_loop(0, _NCHUNK, edge_body, 0)
    plsc.subcore_barrier()

    for t in range(rows // _CH):
        base = sid * rows + t * _CH
        pltpu.sync_copy(acc_sp.at[pl.ds(base, _CH)], ones_v)
        pltpu.sync_copy(ones_v, out_hbm.at[cid, pl.ds(base, _CH)])


@functools.partial(
    pl.kernel,
    out_type=jax.ShapeDtypeStruct((_NC, _NPAD, _F), jnp.float32),
    mesh=_mesh,
    scratch_types=[
        pltpu.VMEM((_CH,), jnp.int32),
        pltpu.VMEM((_CH,), jnp.int32),
        pltpu.VMEM((_CH, _F), jnp.float32),
        pltpu.VMEM_SHARED((_NPAD, _F), jnp.float32),
        pltpu.SemaphoreType.DMA,
    ],
)
def _prop_kernel(g_hbm, src_hbm, dst_hbm, out_hbm, sch, dch, rows_v, acc_sp, sem):
    cid = lax.axis_index("c")
    sid = lax.axis_index("s")
    wid = cid * _NS + sid
    rows = _NPAD // _NS

    zeros16 = jnp.zeros((16,), jnp.float32)

    def zero_body(i, _):
        r = i >> 3
        c = i & 7
        rows_v[r, pl.ds(c * 16, 16)] = zeros16
        return 0

    lax.fori_loop(0, _CH * (_F // 16), zero_body, 0)
    for t in range(rows // _CH):
        pltpu.sync_copy(rows_v, acc_sp.at[pl.ds(sid * rows + t * _CH, _CH)])
    plsc.subcore_barrier()

    def edge_body(j, _):
        base = wid * _EPT + j * _CH
        pltpu.sync_copy(src_hbm.at[pl.ds(base, _CH)], sch)
        pltpu.sync_copy(dst_hbm.at[pl.ds(base, _CH)], dch)
        pltpu.async_copy(g_hbm.at[sch], rows_v, sem).wait()
        pltpu.sync_copy(rows_v, acc_sp.at[dch], add=True)
        return 0

    lax.fori_loop(0, _NCHUNK, edge_body, 0)
    plsc.subcore_barrier()

    for t in range(rows // _CH):
        base = sid * rows + t * _CH
        pltpu.sync_copy(acc_sp.at[pl.ds(base, _CH)], rows_v)
        pltpu.sync_copy(rows_v, out_hbm.at[cid, pl.ds(base, _CH)])


def _prop(g, src_f, dst_f):
    return _prop_kernel(g, src_f, dst_f)


def _dinv_call(degp):
    def body(p_ref, d1_ref, d2_ref):
        deg = p_ref[0] + p_ref[1]
        d = jnp.where(deg > 0, lax.rsqrt(jnp.maximum(deg, 1.0)), 0.0)
        d1_ref[...] = d
        d2_ref[...] = d * d

    return pl.pallas_call(
        body,
        grid=(_NPAD // 2048,),
        in_specs=[pl.BlockSpec((_NC, 2048, _F), lambda i: (0, i, 0))],
        out_specs=(
            pl.BlockSpec((2048, _F), lambda i: (i, 0)),
            pl.BlockSpec((2048, _F), lambda i: (i, 0)),
        ),
        out_shape=(
            jax.ShapeDtypeStruct((_NPAD, _F), jnp.float32),
            jax.ShapeDtypeStruct((_NPAD, _F), jnp.float32),
        ),
    )(degp)


_BN = 2000


def _scale_call(x, d1):
    def body(x_ref, d_ref, o_ref):
        o_ref[...] = x_ref[...] * d_ref[...]

    return pl.pallas_call(
        body,
        grid=(_N // _BN,),
        in_specs=[
            pl.BlockSpec((_BN, _F), lambda i: (i, 0)),
            pl.BlockSpec((_BN, 1), lambda i: (i, 0)),
        ],
        out_specs=pl.BlockSpec((_BN, _F), lambda i: (i, 0)),
        out_shape=jax.ShapeDtypeStruct((_N, _F), jnp.float32),
    )(x, d1)


def _step0_call(s, d1, d2):
    def body(s_ref, d1_ref, d2_ref, tx_ref, g_ref):
        ssum = s_ref[0] + s_ref[1]
        tx_ref[...] = -d1_ref[...] * ssum
        g_ref[...] = -d2_ref[...] * ssum

    return pl.pallas_call(
        body,
        grid=(_N // _BN,),
        in_specs=[
            pl.BlockSpec((_NC, _BN, _F), lambda i: (0, i, 0)),
            pl.BlockSpec((_BN, 1), lambda i: (i, 0)),
            pl.BlockSpec((_BN, 1), lambda i: (i, 0)),
        ],
        out_specs=(
            pl.BlockSpec((_BN, _F), lambda i: (i, 0)),
            pl.BlockSpec((_BN, _F), lambda i: (i, 0)),
        ),
        out_shape=(
            jax.ShapeDtypeStruct((_N, _F), jnp.float32),
            jax.ShapeDtypeStruct((_N, _F), jnp.float32),
        ),
    )(s, d1, d2)


def _stepk_call(s, d1, d2, tx_prev, g_prev):
    def body(s_ref, d1_ref, d2_ref, tp_ref, gp_ref, tx_ref, g_ref):
        ssum = s_ref[0] + s_ref[1]
        tx_ref[...] = -2.0 * d1_ref[...] * ssum - tp_ref[...]
        g_ref[...] = -2.0 * d2_ref[...] * ssum - gp_ref[...]

    return pl.pallas_call(
        body,
        grid=(_N // _BN,),
        in_specs=[
            pl.BlockSpec((_NC, _BN, _F), lambda i: (0, i, 0)),
            pl.BlockSpec((_BN, 1), lambda i: (i, 0)),
            pl.BlockSpec((_BN, 1), lambda i: (i, 0)),
            pl.BlockSpec((_BN, _F), lambda i: (i, 0)),
            pl.BlockSpec((_BN, _F), lambda i: (i, 0)),
        ],
        out_specs=(
            pl.BlockSpec((_BN, _F), lambda i: (i, 0)),
            pl.BlockSpec((_BN, _F), lambda i: (i, 0)),
        ),
        out_shape=(
            jax.ShapeDtypeStruct((_N, _F), jnp.float32),
            jax.ShapeDtypeStruct((_N, _F), jnp.float32),
        ),
    )(s, d1, d2, tx_prev, g_prev)


_BM = 1000


def _matmul_call(x, tx1, tx2, tx3, s3, d1, W, b):
    def body(x_ref, t1_ref, t2_ref, t3_ref, s3_ref, d_ref, w_ref, b_ref, o_ref):
        tx4 = -2.0 * d_ref[...] * (s3_ref[0] + s3_ref[1]) - t2_ref[...]
        acc = jnp.dot(x_ref[...], w_ref[0], preferred_element_type=jnp.float32)
        acc += jnp.dot(t1_ref[...], w_ref[1], preferred_element_type=jnp.float32)
        acc += jnp.dot(t2_ref[...], w_ref[2], preferred_element_type=jnp.float32)
        acc += jnp.dot(t3_ref[...], w_ref[3], preferred_element_type=jnp.float32)
        acc += jnp.dot(tx4, w_ref[4], preferred_element_type=jnp.float32)
        o_ref[...] = acc + b_ref[...]

    return pl.pallas_call(
        body,
        grid=(_N // _BM,),
        in_specs=[
            pl.BlockSpec((_BM, _F), lambda i: (i, 0)),
            pl.BlockSpec((_BM, _F), lambda i: (i, 0)),
            pl.BlockSpec((_BM, _F), lambda i: (i, 0)),
            pl.BlockSpec((_BM, _F), lambda i: (i, 0)),
            pl.BlockSpec((_NC, _BM, _F), lambda i: (0, i, 0)),
            pl.BlockSpec((_BM, 1), lambda i: (i, 0)),
            pl.BlockSpec((_K, _F, _F), lambda i: (0, 0, 0)),
            pl.BlockSpec((1, _F), lambda i: (0, 0)),
        ],
        out_specs=pl.BlockSpec((_BM, _F), lambda i: (i, 0)),
        out_shape=jax.ShapeDtypeStruct((_N, _F), jnp.float32),
    )(x, tx1, tx2, tx3, s3, d1, W, b)


def kernel(x, adj, W, b):
    assert x.shape == (_N, _F) and adj.shape == (2, _E) and W.shape[0] == _K
    adj = adj.astype(jnp.int32)
    src_f = adj[0]
    dst_f = adj[1]

    degp = _deg_kernel(dst_f)
    d1_full, d2_full = _dinv_call(degp)
    d1 = d1_full[:_N, 0:1]
    d2 = d2_full[:_N, 0:1]

    g0 = _scale_call(x, d1)
    s0 = _prop(g0, src_f, dst_f)
    tx1, g1 = _step0_call(s0, d1, d2)
    s1 = _prop(g1, src_f, dst_f)
    tx2, g2 = _stepk_call(s1, d1, d2, x, g0)
    s2 = _prop(g2, src_f, dst_f)
    tx3, g3 = _stepk_call(s2, d1, d2, tx1, g1)
    s3 = _prop(g3, src_f, dst_f)
    out = _matmul_call(x, tx1, tx2, tx3, s3, d1, W, b.reshape(1, _F))
    return out


# speedup vs baseline: 11.5179x; 1.6215x over previous
import functools

import jax
import jax.numpy as jnp
from jax import lax
from jax.experimental import pallas as pl
from jax.experimental.pallas import tpu as pltpu
from jax.experimental.pallas import tpu_sc as plsc

_N = 10000
_E = 320000
_F = 128
_K = 5

_NC = 2
_NS = 16
_NW = _NC * _NS
_EPT = _E // _NW
_CH = 80
_NCHUNK = _EPT // _CH
_NPAD = 10240
_DW = 32

_mesh = plsc.VectorSubcoreMesh(core_axis_name="c", subcore_axis_name="s")


@functools.partial(
    pl.kernel,
    out_type=jax.ShapeDtypeStruct((_NC, _NPAD, _F), jnp.float32),
    mesh=_mesh,
    scratch_types=[
        pltpu.VMEM((_CH,), jnp.int32),
        pltpu.VMEM((_CH,), jnp.int32),
        pltpu.VMEM((_CH, _F), jnp.float32),
        pltpu.VMEM_SHARED((_NPAD, _F), jnp.float32),
        pltpu.SemaphoreType.DMA,
        pltpu.SemaphoreType.DMA,
    ],
)
def _deg_kernel(dst_hbm, out_hbm, dch_a, dch_b, ones_v, acc_sp, sem_da, sem_db):
    cid = lax.axis_index("c")
    sid = lax.axis_index("s")
    wid = cid * _NS + sid
    rows = _NPAD // _NS

    zeros16 = jnp.zeros((16,), jnp.float32)
    ones16 = jnp.ones((16,), jnp.float32)

    def zero_body(i, _):
        ones_v[i >> 3, pl.ds((i & 7) * 16, 16)] = zeros16
        return 0

    lax.fori_loop(0, _CH * (_F // 16), zero_body, 0)
    for t in range(rows // _CH):
        pltpu.sync_copy(ones_v, acc_sp.at[pl.ds(sid * rows + t * _CH, _CH)])

    def ones_body(i, _):
        ones_v[i >> 3, pl.ds((i & 7) * 16, 16)] = ones16
        return 0

    lax.fori_loop(0, _CH * (_F // 16), ones_body, 0)
    plsc.subcore_barrier()

    def didx(j, buf, sem):
        return pltpu.make_async_copy(
            dst_hbm.at[pl.ds(wid * _EPT + j * _CH, _CH)], buf, sem)

    didx(0, dch_a, sem_da).start()
    didx(1, dch_b, sem_db).start()

    def pair_body(t, _):
        ja = 2 * t
        jb = 2 * t + 1
        jn = 2 * t + 2
        jn1 = jnp.minimum(jn + 1, _NCHUNK - 1)
        didx(ja, dch_a, sem_da).wait()
        pltpu.sync_copy(ones_v, acc_sp.at[dch_a], add=True)
        didx(jn, dch_a, sem_da).start()
        didx(jb, dch_b, sem_db).wait()
        pltpu.sync_copy(ones_v, acc_sp.at[dch_b], add=True)
        didx(jn1, dch_b, sem_db).start()
        return 0

    lax.fori_loop(0, (_NCHUNK - 1) // 2, pair_body, 0)
    last = _NCHUNK - 1
    didx(last, dch_a, sem_da).wait()
    pltpu.sync_copy(ones_v, acc_sp.at[dch_a], add=True)
    didx(last, dch_b, sem_db).wait()
    plsc.subcore_barrier()

    for t in range(rows // _CH):
        base = sid * rows + t * _CH
        pltpu.sync_copy(acc_sp.at[pl.ds(base, _CH)], ones_v)
        pltpu.sync_copy(ones_v, out_hbm.at[cid, pl.ds(base, _CH)])


@functools.partial(
    pl.kernel,
    out_type=jax.ShapeDtypeStruct((_NC, _NPAD, _F), jnp.float32),
    mesh=_mesh,
    scratch_types=[
        pltpu.VMEM((_CH,), jnp.int32),
        pltpu.VMEM((_CH,), jnp.int32),
        pltpu.VMEM((_CH,), jnp.int32),
        pltpu.VMEM((_CH,), jnp.int32),
        pltpu.VMEM((_CH, _F), jnp.float32),
        pltpu.VMEM((_CH, _F), jnp.float32),
        pltpu.VMEM_SHARED((_NPAD, _F), jnp.float32),
        pltpu.SemaphoreType.DMA,
        pltpu.SemaphoreType.DMA,
        pltpu.SemaphoreType.DMA,
        pltpu.SemaphoreType.DMA,
        pltpu.SemaphoreType.DMA,
        pltpu.SemaphoreType.DMA,
    ],
)
def _prop_kernel(g_hbm, src_hbm, dst_hbm, out_hbm,
                 sch_a, sch_b, dch_a, dch_b, rows_a, rows_b, acc_sp,
                 sem_sa, sem_sb, sem_da, sem_db, sem_ga, sem_gb):
    cid = lax.axis_index("c")
    sid = lax.axis_index("s")
    wid = cid * _NS + sid
    rows = _NPAD // _NS
    ebase = wid * _EPT

    zeros16 = jnp.zeros((16,), jnp.float32)

    def zero_body(i, _):
        rows_a[i >> 3, pl.ds((i & 7) * 16, 16)] = zeros16
        return 0

    lax.fori_loop(0, _CH * (_F // 16), zero_body, 0)
    for t in range(rows // _CH):
        pltpu.sync_copy(rows_a, acc_sp.at[pl.ds(sid * rows + t * _CH, _CH)])
    plsc.subcore_barrier()

    def sidx(j, buf, sem):
        return pltpu.make_async_copy(
            src_hbm.at[pl.ds(ebase + j * _CH, _CH)], buf, sem)

    def didx(j, buf, sem):
        return pltpu.make_async_copy(
            dst_hbm.at[pl.ds(ebase + j * _CH, _CH)], buf, sem)

    def gath(buf_idx, buf_rows, sem):
        return pltpu.make_async_copy(g_hbm.at[buf_idx], buf_rows, sem)

    sidx(0, sch_a, sem_sa).start()
    didx(0, dch_a, sem_da).start()
    sidx(1, sch_b, sem_sb).start()
    didx(1, dch_b, sem_db).start()
    sidx(0, sch_a, sem_sa).wait()
    gath(sch_a, rows_a, sem_ga).start()

    def pair_body(t, _):
        ja = 2 * t
        jb = 2 * t + 1
        jn = 2 * t + 2
        gath(sch_a, rows_a, sem_ga).wait()
        sidx(jn, sch_a, sem_sa).start()
        sidx(jb, sch_b, sem_sb).wait()
        gath(sch_b, rows_b, sem_gb).start()
        didx(ja, dch_a, sem_da).wait()
        pltpu.sync_copy(rows_a, acc_sp.at[dch_a], add=True)
        didx(jn, dch_a, sem_da).start()
        jn1 = jnp.minimum(jn + 1, _NCHUNK - 1)
        gath(sch_b, rows_b, sem_gb).wait()
        sidx(jn1, sch_b, sem_sb).start()
        didx(jb, dch_b, sem_db).wait()
        pltpu.sync_copy(rows_b, acc_sp.at[dch_b], add=True)
        didx(jn1, dch_b, sem_db).start()
        sidx(jn, sch_a, sem_sa).wait()
        gath(sch_a, rows_a, sem_ga).start()
        return 0

    lax.fori_loop(0, (_NCHUNK - 1) // 2, pair_body, 0)
    last = _NCHUNK - 1
    gath(sch_a, rows_a, sem_ga).wait()
    didx(last, dch_a, sem_da).wait()
    pltpu.sync_copy(rows_a, acc_sp.at[dch_a], add=True)
    sidx(last, sch_b, sem_sb).wait()
    didx(last, dch_b, sem_db).wait()

    plsc.subcore_barrier()
    for t in range(rows // _CH):
        base = sid * rows + t * _CH
        pltpu.sync_copy(acc_sp.at[pl.ds(base, _CH)], rows_a)
        pltpu.sync_copy(rows_a, out_hbm.at[cid, pl.ds(base, _CH)])


def _prop(g, src_f, dst_f):
    return _prop_kernel(g, src_f, dst_f)


def _dinv_call(degp):
    def body(p_ref, d1_ref, d2_ref):
        deg = p_ref[0] + p_ref[1]
        d = jnp.where(deg > 0, lax.rsqrt(jnp.maximum(deg, 1.0)), 0.0)
        d1_ref[...] = d
        d2_ref[...] = d * d

    return pl.pallas_call(
        body,
        grid=(_NPAD // 2048,),
        in_specs=[pl.BlockSpec((_NC, 2048, _F), lambda i: (0, i, 0))],
        out_specs=(
            pl.BlockSpec((2048, _F), lambda i: (i, 0)),
            pl.BlockSpec((2048, _F), lambda i: (i, 0)),
        ),
        out_shape=(
            jax.ShapeDtypeStruct((_NPAD, _F), jnp.float32),
            jax.ShapeDtypeStruct((_NPAD, _F), jnp.float32),
        ),
    )(degp)


_BN = 2000


def _scale_call(x, d1):
    def body(x_ref, d_ref, o_ref):
        o_ref[...] = x_ref[...] * d_ref[...]

    return pl.pallas_call(
        body,
        grid=(_N // _BN,),
        in_specs=[
            pl.BlockSpec((_BN, _F), lambda i: (i, 0)),
            pl.BlockSpec((_BN, 1), lambda i: (i, 0)),
        ],
        out_specs=pl.BlockSpec((_BN, _F), lambda i: (i, 0)),
        out_shape=jax.ShapeDtypeStruct((_N, _F), jnp.float32),
    )(x, d1)


def _step0_call(s, d1, d2):
    def body(s_ref, d1_ref, d2_ref, tx_ref, g_ref):
        ssum = s_ref[0] + s_ref[1]
        tx_ref[...] = -d1_ref[...] * ssum
        g_ref[...] = -d2_ref[...] * ssum

    return pl.pallas_call(
        body,
        grid=(_N // _BN,),
        in_specs=[
            pl.BlockSpec((_NC, _BN, _F), lambda i: (0, i, 0)),
            pl.BlockSpec((_BN, 1), lambda i: (i, 0)),
            pl.BlockSpec((_BN, 1), lambda i: (i, 0)),
        ],
        out_specs=(
            pl.BlockSpec((_BN, _F), lambda i: (i, 0)),
            pl.BlockSpec((_BN, _F), lambda i: (i, 0)),
        ),
        out_shape=(
            jax.ShapeDtypeStruct((_N, _F), jnp.float32),
            jax.ShapeDtypeStruct((_N, _F), jnp.float32),
        ),
    )(s, d1, d2)


def _stepk_call(s, d1, d2, tx_prev, g_prev):
    def body(s_ref, d1_ref, d2_ref, tp_ref, gp_ref, tx_ref, g_ref):
        ssum = s_ref[0] + s_ref[1]
        tx_ref[...] = -2.0 * d1_ref[...] * ssum - tp_ref[...]
        g_ref[...] = -2.0 * d2_ref[...] * ssum - gp_ref[...]

    return pl.pallas_call(
        body,
        grid=(_N // _BN,),
        in_specs=[
            pl.BlockSpec((_NC, _BN, _F), lambda i: (0, i, 0)),
            pl.BlockSpec((_BN, 1), lambda i: (i, 0)),
            pl.BlockSpec((_BN, 1), lambda i: (i, 0)),
            pl.BlockSpec((_BN, _F), lambda i: (i, 0)),
            pl.BlockSpec((_BN, _F), lambda i: (i, 0)),
        ],
        out_specs=(
            pl.BlockSpec((_BN, _F), lambda i: (i, 0)),
            pl.BlockSpec((_BN, _F), lambda i: (i, 0)),
        ),
        out_shape=(
            jax.ShapeDtypeStruct((_N, _F), jnp.float32),
            jax.ShapeDtypeStruct((_N, _F), jnp.float32),
        ),
    )(s, d1, d2, tx_prev, g_prev)


_BM = 1000


def _matmul_call(x, tx1, tx2, tx3, s3, d1, W, b):
    def body(x_ref, t1_ref, t2_ref, t3_ref, s3_ref, d_ref, w_ref, b_ref, o_ref):
        tx4 = -2.0 * d_ref[...] * (s3_ref[0] + s3_ref[1]) - t2_ref[...]
        acc = jnp.dot(x_ref[...], w_ref[0], preferred_element_type=jnp.float32)
        acc += jnp.dot(t1_ref[...], w_ref[1], preferred_element_type=jnp.float32)
        acc += jnp.dot(t2_ref[...], w_ref[2], preferred_element_type=jnp.float32)
        acc += jnp.dot(t3_ref[...], w_ref[3], preferred_element_type=jnp.float32)
        acc += jnp.dot(tx4, w_ref[4], preferred_element_type=jnp.float32)
        o_ref[...] = acc + b_ref[...]

    return pl.pallas_call(
        body,
        grid=(_N // _BM,),
        in_specs=[
            pl.BlockSpec((_BM, _F), lambda i: (i, 0)),
            pl.BlockSpec((_BM, _F), lambda i: (i, 0)),
            pl.BlockSpec((_BM, _F), lambda i: (i, 0)),
            pl.BlockSpec((_BM, _F), lambda i: (i, 0)),
            pl.BlockSpec((_NC, _BM, _F), lambda i: (0, i, 0)),
            pl.BlockSpec((_BM, 1), lambda i: (i, 0)),
            pl.BlockSpec((_K, _F, _F), lambda i: (0, 0, 0)),
            pl.BlockSpec((1, _F), lambda i: (0, 0)),
        ],
        out_specs=pl.BlockSpec((_BM, _F), lambda i: (i, 0)),
        out_shape=jax.ShapeDtypeStruct((_N, _F), jnp.float32),
    )(x, tx1, tx2, tx3, s3, d1, W, b)


def kernel(x, adj, W, b):
    assert x.shape == (_N, _F) and adj.shape == (2, _E) and W.shape[0] == _K
    adj = adj.astype(jnp.int32)
    src_f = adj[0]
    dst_f = adj[1]

    degp = _deg_kernel(dst_f)
    d1_full, d2_full = _dinv_call(degp)
    d1 = d1_full[:_N, 0:1]
    d2 = d2_full[:_N, 0:1]

    g0 = _scale_call(x, d1)
    s0 = _prop(g0, src_f, dst_f)
    tx1, g1 = _step0_call(s0, d1, d2)
    s1 = _prop(g1, src_f, dst_f)
    tx2, g2 = _stepk_call(s1, d1, d2, x, g0)
    s2 = _prop(g2, src_f, dst_f)
    tx3, g3 = _stepk_call(s2, d1, d2, tx1, g1)
    s3 = _prop(g3, src_f, dst_f)
    out = _matmul_call(x, tx1, tx2, tx3, s3, d1, W, b.reshape(1, _F))
    return out


# speedup vs baseline: 15.4545x; 1.3418x over previous
import functools

import jax
import jax.numpy as jnp
from jax import lax
from jax.experimental import pallas as pl
from jax.experimental.pallas import tpu as pltpu
from jax.experimental.pallas import tpu_sc as plsc

_N = 10000
_E = 320000
_F = 128
_K = 5

_NC = 2
_NS = 16
_NW = _NC * _NS
_EPT = _E // _NW
_CH = 80
_NCHUNK = _EPT // _CH
_NPAD = 10240
_DW = 32

_mesh = plsc.VectorSubcoreMesh(core_axis_name="c", subcore_axis_name="s")


@functools.partial(
    pl.kernel,
    out_type=jax.ShapeDtypeStruct((_NC, _NPAD, _F), jnp.float32),
    mesh=_mesh,
    scratch_types=[
        pltpu.VMEM((_CH,), jnp.int32),
        pltpu.VMEM((_CH,), jnp.int32),
        pltpu.VMEM((_CH, _F), jnp.float32),
        pltpu.VMEM_SHARED((_NPAD, _F), jnp.float32),
        pltpu.SemaphoreType.DMA,
        pltpu.SemaphoreType.DMA,
    ],
)
def _deg_kernel(dst_hbm, out_hbm, dch_a, dch_b, ones_v, acc_sp, sem_da, sem_db):
    cid = lax.axis_index("c")
    sid = lax.axis_index("s")
    wid = cid * _NS + sid
    rows = _NPAD // _NS

    zeros16 = jnp.zeros((16,), jnp.float32)
    ones16 = jnp.ones((16,), jnp.float32)

    def zero_body(i, _):
        ones_v[i >> 3, pl.ds((i & 7) * 16, 16)] = zeros16
        return 0

    lax.fori_loop(0, _CH * (_F // 16), zero_body, 0)
    for t in range(rows // _CH):
        pltpu.sync_copy(ones_v, acc_sp.at[pl.ds(sid * rows + t * _CH, _CH)])

    def ones_body(i, _):
        ones_v[i >> 3, pl.ds((i & 7) * 16, 16)] = ones16
        return 0

    lax.fori_loop(0, _CH * (_F // 16), ones_body, 0)
    plsc.subcore_barrier()

    def didx(j, buf, sem):
        return pltpu.make_async_copy(
            dst_hbm.at[pl.ds(wid * _EPT + j * _CH, _CH)], buf, sem)

    didx(0, dch_a, sem_da).start()
    didx(1, dch_b, sem_db).start()

    def pair_body(t, _):
        ja = 2 * t
        jb = 2 * t + 1
        jn = 2 * t + 2
        jn1 = jnp.minimum(jn + 1, _NCHUNK - 1)
        didx(ja, dch_a, sem_da).wait()
        pltpu.sync_copy(ones_v, acc_sp.at[dch_a], add=True)
        didx(jn, dch_a, sem_da).start()
        didx(jb, dch_b, sem_db).wait()
        pltpu.sync_copy(ones_v, acc_sp.at[dch_b], add=True)
        didx(jn1, dch_b, sem_db).start()
        return 0

    lax.fori_loop(0, (_NCHUNK - 1) // 2, pair_body, 0)
    last = _NCHUNK - 1
    didx(last, dch_a, sem_da).wait()
    pltpu.sync_copy(ones_v, acc_sp.at[dch_a], add=True)
    didx(last, dch_b, sem_db).wait()
    plsc.subcore_barrier()

    for t in range(rows // _CH):
        base = sid * rows + t * _CH
        pltpu.sync_copy(acc_sp.at[pl.ds(base, _CH)], ones_v)
        pltpu.sync_copy(ones_v, out_hbm.at[cid, pl.ds(base, _CH)])


@functools.partial(
    pl.kernel,
    out_type=jax.ShapeDtypeStruct((_NC, _NPAD, _F), jnp.float32),
    mesh=_mesh,
    scratch_types=(
        [pltpu.VMEM((_CH,), jnp.int32) for _ in range(3)]
        + [pltpu.VMEM((_CH,), jnp.int32) for _ in range(3)]
        + [pltpu.VMEM((_CH, _F), jnp.float32) for _ in range(3)]
        + [pltpu.VMEM_SHARED((_NPAD, _F), jnp.float32)]
        + [pltpu.SemaphoreType.DMA for _ in range(12)]
    ),
)
def _prop_kernel(g_hbm, src_hbm, dst_hbm, out_hbm,
                 sch0, sch1, sch2, dch0, dch1, dch2, rw0, rw1, rw2, acc_sp,
                 ss0, ss1, ss2, sd0, sd1, sd2, sg0, sg1, sg2, sc0, sc1, sc2):
    cid = lax.axis_index("c")
    sid = lax.axis_index("s")
    wid = cid * _NS + sid
    rows = _NPAD // _NS
    ebase = wid * _EPT
    last = _NCHUNK - 1

    sch = [sch0, sch1, sch2]
    dch = [dch0, dch1, dch2]
    rw = [rw0, rw1, rw2]
    ss = [ss0, ss1, ss2]
    sd = [sd0, sd1, sd2]
    sg = [sg0, sg1, sg2]
    sc = [sc0, sc1, sc2]

    zeros16 = jnp.zeros((16,), jnp.float32)

    def zero_body(i, _):
        rw0[i >> 3, pl.ds((i & 7) * 16, 16)] = zeros16
        return 0

    lax.fori_loop(0, _CH * (_F // 16), zero_body, 0)
    for t in range(rows // _CH):
        pltpu.sync_copy(rw0, acc_sp.at[pl.ds(sid * rows + t * _CH, _CH)])
    plsc.subcore_barrier()

    def sidx(j, x):
        return pltpu.make_async_copy(
            src_hbm.at[pl.ds(ebase + j * _CH, _CH)], sch[x], ss[x])

    def didx(j, x):
        return pltpu.make_async_copy(
            dst_hbm.at[pl.ds(ebase + j * _CH, _CH)], dch[x], sd[x])

    def gath(x):
        return pltpu.make_async_copy(g_hbm.at[sch[x]], rw[x], sg[x])

    def scat_start(x):
        pltpu.async_copy(rw[x], acc_sp.at[dch[x]], sc[x], add=True)

    def scat_wait(x):
        pltpu.make_async_copy(rw[x], acc_sp.at[dch[x]], sc[x]).wait()

    for x in range(3):
        sidx(x, x).start()
        didx(x, x).start()
    for x in range(3):
        sidx(x, x).wait()
        gath(x).start()

    def body(t, _):
        j3 = 3 * t
        for x in range(3):
            gath(x).wait()
            sidx(jnp.minimum(j3 + 3 + x, last), x).start()
            didx(j3 + x, x).wait()
            scat_start(x)
        for x in range(3):
            scat_wait(x)
            didx(jnp.minimum(j3 + 3 + x, last), x).start()
            sidx(jnp.minimum(j3 + 3 + x, last), x).wait()
            gath(x).start()
        return 0

    nbody = (_NCHUNK - 2) // 3
    lax.fori_loop(0, nbody, body, 0)
    gath(0).wait()
    didx(last - 1, 0).wait()
    scat_start(0)
    gath(1).wait()
    didx(last, 1).wait()
    scat_start(1)
    gath(2).wait()
    didx(last, 2).wait()
    scat_wait(0)
    scat_wait(1)

    plsc.subcore_barrier()
    for t in range(rows // _CH):
        base = sid * rows + t * _CH
        pltpu.sync_copy(acc_sp.at[pl.ds(base, _CH)], rw0)
        pltpu.sync_copy(rw0, out_hbm.at[cid, pl.ds(base, _CH)])


def _prop(g, src_f, dst_f):
    return _prop_kernel(g, src_f, dst_f)


def _dinv_call(degp):
    def body(p_ref, d1_ref, d2_ref):
        deg = p_ref[0] + p_ref[1]
        d = jnp.where(deg > 0, lax.rsqrt(jnp.maximum(deg, 1.0)), 0.0)
        d1_ref[...] = d
        d2_ref[...] = d * d

    return pl.pallas_call(
        body,
        grid=(_NPAD // 2048,),
        in_specs=[pl.BlockSpec((_NC, 2048, _F), lambda i: (0, i, 0))],
        out_specs=(
            pl.BlockSpec((2048, _F), lambda i: (i, 0)),
            pl.BlockSpec((2048, _F), lambda i: (i, 0)),
        ),
        out_shape=(
            jax.ShapeDtypeStruct((_NPAD, _F), jnp.float32),
            jax.ShapeDtypeStruct((_NPAD, _F), jnp.float32),
        ),
    )(degp)


_BN = 2000


def _scale_call(x, d1):
    def body(x_ref, d_ref, o_ref):
        o_ref[...] = x_ref[...] * d_ref[...]

    return pl.pallas_call(
        body,
        grid=(_N // _BN,),
        in_specs=[
            pl.BlockSpec((_BN, _F), lambda i: (i, 0)),
            pl.BlockSpec((_BN, 1), lambda i: (i, 0)),
        ],
        out_specs=pl.BlockSpec((_BN, _F), lambda i: (i, 0)),
        out_shape=jax.ShapeDtypeStruct((_N, _F), jnp.float32),
    )(x, d1)


def _step0_call(s, d1, d2):
    def body(s_ref, d1_ref, d2_ref, tx_ref, g_ref):
        ssum = s_ref[0] + s_ref[1]
        tx_ref[...] = -d1_ref[...] * ssum
        g_ref[...] = -d2_ref[...] * ssum

    return pl.pallas_call(
        body,
        grid=(_N // _BN,),
        in_specs=[
            pl.BlockSpec((_NC, _BN, _F), lambda i: (0, i, 0)),
            pl.BlockSpec((_BN, 1), lambda i: (i, 0)),
            pl.BlockSpec((_BN, 1), lambda i: (i, 0)),
        ],
        out_specs=(
            pl.BlockSpec((_BN, _F), lambda i: (i, 0)),
            pl.BlockSpec((_BN, _F), lambda i: (i, 0)),
        ),
        out_shape=(
            jax.ShapeDtypeStruct((_N, _F), jnp.float32),
            jax.ShapeDtypeStruct((_N, _F), jnp.float32),
        ),
    )(s, d1, d2)


def _stepk_call(s, d1, d2, tx_prev, g_prev):
    def body(s_ref, d1_ref, d2_ref, tp_ref, gp_ref, tx_ref, g_ref):
        ssum = s_ref[0] + s_ref[1]
        tx_ref[...] = -2.0 * d1_ref[...] * ssum - tp_ref[...]
        g_ref[...] = -2.0 * d2_ref[...] * ssum - gp_ref[...]

    return pl.pallas_call(
        body,
        grid=(_N // _BN,),
        in_specs=[
            pl.BlockSpec((_NC, _BN, _F), lambda i: (0, i, 0)),
            pl.BlockSpec((_BN, 1), lambda i: (i, 0)),
            pl.BlockSpec((_BN, 1), lambda i: (i, 0)),
            pl.BlockSpec((_BN, _F), lambda i: (i, 0)),
            pl.BlockSpec((_BN, _F), lambda i: (i, 0)),
        ],
        out_specs=(
            pl.BlockSpec((_BN, _F), lambda i: (i, 0)),
            pl.BlockSpec((_BN, _F), lambda i: (i, 0)),
        ),
        out_shape=(
            jax.ShapeDtypeStruct((_N, _F), jnp.float32),
            jax.ShapeDtypeStruct((_N, _F), jnp.float32),
        ),
    )(s, d1, d2, tx_prev, g_prev)


_BM = 1000


def _matmul_call(x, tx1, tx2, tx3, s3, d1, W, b):
    def body(x_ref, t1_ref, t2_ref, t3_ref, s3_ref, d_ref, w_ref, b_ref, o_ref):
        tx4 = -2.0 * d_ref[...] * (s3_ref[0] + s3_ref[1]) - t2_ref[...]
        acc = jnp.dot(x_ref[...], w_ref[0], preferred_element_type=jnp.float32)
        acc += jnp.dot(t1_ref[...], w_ref[1], preferred_element_type=jnp.float32)
        acc += jnp.dot(t2_ref[...], w_ref[2], preferred_element_type=jnp.float32)
        acc += jnp.dot(t3_ref[...], w_ref[3], preferred_element_type=jnp.float32)
        acc += jnp.dot(tx4, w_ref[4], preferred_element_type=jnp.float32)
        o_ref[...] = acc + b_ref[...]

    return pl.pallas_call(
        body,
        grid=(_N // _BM,),
        in_specs=[
            pl.BlockSpec((_BM, _F), lambda i: (i, 0)),
            pl.BlockSpec((_BM, _F), lambda i: (i, 0)),
            pl.BlockSpec((_BM, _F), lambda i: (i, 0)),
            pl.BlockSpec((_BM, _F), lambda i: (i, 0)),
            pl.BlockSpec((_NC, _BM, _F), lambda i: (0, i, 0)),
            pl.BlockSpec((_BM, 1), lambda i: (i, 0)),
            pl.BlockSpec((_K, _F, _F), lambda i: (0, 0, 0)),
            pl.BlockSpec((1, _F), lambda i: (0, 0)),
        ],
        out_specs=pl.BlockSpec((_BM, _F), lambda i: (i, 0)),
        out_shape=jax.ShapeDtypeStruct((_N, _F), jnp.float32),
    )(x, tx1, tx2, tx3, s3, d1, W, b)


def kernel(x, adj, W, b):
    assert x.shape == (_N, _F) and adj.shape == (2, _E) and W.shape[0] == _K
    adj = adj.astype(jnp.int32)
    src_f = adj[0]
    dst_f = adj[1]

    degp = _deg_kernel(dst_f)
    d1_full, d2_full = _dinv_call(degp)
    d1 = d1_full[:_N, 0:1]
    d2 = d2_full[:_N, 0:1]

    g0 = _scale_call(x, d1)
    s0 = _prop(g0, src_f, dst_f)
    tx1, g1 = _step0_call(s0, d1, d2)
    s1 = _prop(g1, src_f, dst_f)
    tx2, g2 = _stepk_call(s1, d1, d2, x, g0)
    s2 = _prop(g2, src_f, dst_f)
    tx3, g3 = _stepk_call(s2, d1, d2, tx1, g1)
    s3 = _prop(g3, src_f, dst_f)
    out = _matmul_call(x, tx1, tx2, tx3, s3, d1, W, b.reshape(1, _F))
    return out


# speedup vs baseline: 15.6438x; 1.0122x over previous
import functools

import jax
import jax.numpy as jnp
from jax import lax
from jax.experimental import pallas as pl
from jax.experimental.pallas import tpu as pltpu
from jax.experimental.pallas import tpu_sc as plsc

_N = 10000
_E = 320000
_F = 128
_K = 5

_NC = 2
_NS = 16
_NW = _NC * _NS
_EPT = _E // _NW
_CH = 80
_NCHUNK = _EPT // _CH
_NPAD = 10240
_DW = 32

_mesh = plsc.VectorSubcoreMesh(core_axis_name="c", subcore_axis_name="s")


@functools.partial(
    pl.kernel,
    out_type=jax.ShapeDtypeStruct((_NC, _NPAD, _F), jnp.float32),
    mesh=_mesh,
    scratch_types=(
        [pltpu.VMEM((_CH,), jnp.int32) for _ in range(3)]
        + [pltpu.VMEM((_CH, _F), jnp.float32)]
        + [pltpu.VMEM_SHARED((_NPAD, _F), jnp.float32)]
        + [pltpu.SemaphoreType.DMA for _ in range(6)]
    ),
)
def _deg_kernel(dst_hbm, out_hbm, dch0, dch1, dch2, ones_v, acc_sp,
                sd0, sd1, sd2, sc0, sc1, sc2):
    cid = lax.axis_index("c")
    sid = lax.axis_index("s")
    wid = cid * _NS + sid
    rows = _NPAD // _NS
    ebase = wid * _EPT
    last = _NCHUNK - 1
    dch = [dch0, dch1, dch2]
    sd = [sd0, sd1, sd2]
    sc = [sc0, sc1, sc2]

    zeros16 = jnp.zeros((16,), jnp.float32)
    ones16 = jnp.ones((16,), jnp.float32)

    def zero_body(i, _):
        ones_v[i >> 3, pl.ds((i & 7) * 16, 16)] = zeros16
        return 0

    lax.fori_loop(0, _CH * (_F // 16), zero_body, 0)
    for t in range(rows // _CH):
        pltpu.sync_copy(ones_v, acc_sp.at[pl.ds(sid * rows + t * _CH, _CH)])

    def ones_body(i, _):
        ones_v[i >> 3, pl.ds((i & 7) * 16, 16)] = ones16
        return 0

    lax.fori_loop(0, _CH * (_F // 16), ones_body, 0)
    plsc.subcore_barrier()

    def didx(j, x):
        return pltpu.make_async_copy(
            dst_hbm.at[pl.ds(ebase + j * _CH, _CH)], dch[x], sd[x])

    def scat_start(x):
        pltpu.async_copy(ones_v, acc_sp.at[dch[x]], sc[x], add=True)

    def scat_wait(x):
        pltpu.make_async_copy(ones_v, acc_sp.at[dch[x]], sc[x]).wait()

    for x in range(3):
        didx(x, x).start()

    def body(t, _):
        j3 = 3 * t
        for x in range(3):
            didx(j3 + x, x).wait()
            scat_start(x)
        for x in range(3):
            scat_wait(x)
            didx(jnp.minimum(j3 + 3 + x, last), x).start()
        return 0

    nbody = (_NCHUNK - 2) // 3
    lax.fori_loop(0, nbody, body, 0)
    didx(last - 1, 0).wait()
    scat_start(0)
    didx(last, 1).wait()
    scat_start(1)
    didx(last, 2).wait()
    scat_wait(0)
    scat_wait(1)
    plsc.subcore_barrier()

    for t in range(rows // _CH):
        base = sid * rows + t * _CH
        pltpu.sync_copy(acc_sp.at[pl.ds(base, _CH)], out_hbm.at[cid, pl.ds(base, _CH)])


@functools.partial(
    pl.kernel,
    out_type=jax.ShapeDtypeStruct((_NC, _NPAD, _F), jnp.float32),
    mesh=_mesh,
    scratch_types=(
        [pltpu.VMEM((_CH,), jnp.int32) for _ in range(3)]
        + [pltpu.VMEM((_CH,), jnp.int32) for _ in range(3)]
        + [pltpu.VMEM((_CH, _F), jnp.float32) for _ in range(3)]
        + [pltpu.VMEM_SHARED((_NPAD, _F), jnp.float32)]
        + [pltpu.SemaphoreType.DMA for _ in range(12)]
    ),
)
def _prop_kernel(g_hbm, src_hbm, dst_hbm, out_hbm,
                 sch0, sch1, sch2, dch0, dch1, dch2, rw0, rw1, rw2, acc_sp,
                 ss0, ss1, ss2, sd0, sd1, sd2, sg0, sg1, sg2, sc0, sc1, sc2):
    cid = lax.axis_index("c")
    sid = lax.axis_index("s")
    wid = cid * _NS + sid
    rows = _NPAD // _NS
    ebase = wid * _EPT
    last = _NCHUNK - 1

    sch = [sch0, sch1, sch2]
    dch = [dch0, dch1, dch2]
    rw = [rw0, rw1, rw2]
    ss = [ss0, ss1, ss2]
    sd = [sd0, sd1, sd2]
    sg = [sg0, sg1, sg2]
    sc = [sc0, sc1, sc2]

    zeros16 = jnp.zeros((16,), jnp.float32)

    def zero_body(i, _):
        rw0[i >> 3, pl.ds((i & 7) * 16, 16)] = zeros16
        return 0

    lax.fori_loop(0, _CH * (_F // 16), zero_body, 0)
    for t in range(rows // _CH):
        pltpu.sync_copy(rw0, acc_sp.at[pl.ds(sid * rows + t * _CH, _CH)])
    plsc.subcore_barrier()

    def sidx(j, x):
        return pltpu.make_async_copy(
            src_hbm.at[pl.ds(ebase + j * _CH, _CH)], sch[x], ss[x])

    def didx(j, x):
        return pltpu.make_async_copy(
            dst_hbm.at[pl.ds(ebase + j * _CH, _CH)], dch[x], sd[x])

    def gath(x):
        return pltpu.make_async_copy(g_hbm.at[sch[x]], rw[x], sg[x])

    def scat_start(x):
        pltpu.async_copy(rw[x], acc_sp.at[dch[x]], sc[x], add=True)

    def scat_wait(x):
        pltpu.make_async_copy(rw[x], acc_sp.at[dch[x]], sc[x]).wait()

    for x in range(3):
        sidx(x, x).start()
        didx(x, x).start()
    for x in range(3):
        sidx(x, x).wait()
        gath(x).start()

    def body(t, _):
        j3 = 3 * t
        for x in range(3):
            gath(x).wait()
            sidx(jnp.minimum(j3 + 3 + x, last), x).start()
            didx(j3 + x, x).wait()
            scat_start(x)
        for x in range(3):
            scat_wait(x)
            didx(jnp.minimum(j3 + 3 + x, last), x).start()
            sidx(jnp.minimum(j3 + 3 + x, last), x).wait()
            gath(x).start()
        return 0

    nbody = (_NCHUNK - 2) // 3
    lax.fori_loop(0, nbody, body, 0)
    gath(0).wait()
    didx(last - 1, 0).wait()
    scat_start(0)
    gath(1).wait()
    didx(last, 1).wait()
    scat_start(1)
    gath(2).wait()
    didx(last, 2).wait()
    scat_wait(0)
    scat_wait(1)

    plsc.subcore_barrier()
    for t in range(rows // _CH):
        base = sid * rows + t * _CH
        pltpu.sync_copy(acc_sp.at[pl.ds(base, _CH)], out_hbm.at[cid, pl.ds(base, _CH)])


def _prop(g, src_f, dst_f):
    return _prop_kernel(g, src_f, dst_f)


def _dinv_call(degp):
    def body(p_ref, d1_ref, d2_ref):
        deg = p_ref[0] + p_ref[1]
        d = jnp.where(deg > 0, lax.rsqrt(jnp.maximum(deg, 1.0)), 0.0)
        d1_ref[...] = d
        d2_ref[...] = d * d

    return pl.pallas_call(
        body,
        grid=(_NPAD // 2048,),
        in_specs=[pl.BlockSpec((_NC, 2048, _F), lambda i: (0, i, 0))],
        out_specs=(
            pl.BlockSpec((2048, _F), lambda i: (i, 0)),
            pl.BlockSpec((2048, _F), lambda i: (i, 0)),
        ),
        out_shape=(
            jax.ShapeDtypeStruct((_NPAD, _F), jnp.float32),
            jax.ShapeDtypeStruct((_NPAD, _F), jnp.float32),
        ),
    )(degp)


_BN = 2000


def _scale_call(x, d1):
    def body(x_ref, d_ref, o_ref):
        o_ref[...] = x_ref[...] * d_ref[...]

    return pl.pallas_call(
        body,
        grid=(_N // _BN,),
        in_specs=[
            pl.BlockSpec((_BN, _F), lambda i: (i, 0)),
            pl.BlockSpec((_BN, 1), lambda i: (i, 0)),
        ],
        out_specs=pl.BlockSpec((_BN, _F), lambda i: (i, 0)),
        out_shape=jax.ShapeDtypeStruct((_N, _F), jnp.float32),
    )(x, d1)


def _step0_call(s, d1, d2):
    def body(s_ref, d1_ref, d2_ref, tx_ref, g_ref):
        ssum = s_ref[0] + s_ref[1]
        tx_ref[...] = -d1_ref[...] * ssum
        g_ref[...] = -d2_ref[...] * ssum

    return pl.pallas_call(
        body,
        grid=(_N // _BN,),
        in_specs=[
            pl.BlockSpec((_NC, _BN, _F), lambda i: (0, i, 0)),
            pl.BlockSpec((_BN, 1), lambda i: (i, 0)),
            pl.BlockSpec((_BN, 1), lambda i: (i, 0)),
        ],
        out_specs=(
            pl.BlockSpec((_BN, _F), lambda i: (i, 0)),
            pl.BlockSpec((_BN, _F), lambda i: (i, 0)),
        ),
        out_shape=(
            jax.ShapeDtypeStruct((_N, _F), jnp.float32),
            jax.ShapeDtypeStruct((_N, _F), jnp.float32),
        ),
    )(s, d1, d2)


def _stepk_call(s, d1, d2, tx_prev, g_prev):
    def body(s_ref, d1_ref, d2_ref, tp_ref, gp_ref, tx_ref, g_ref):
        ssum = s_ref[0] + s_ref[1]
        tx_ref[...] = -2.0 * d1_ref[...] * ssum - tp_ref[...]
        g_ref[...] = -2.0 * d2_ref[...] * ssum - gp_ref[...]

    return pl.pallas_call(
        body,
        grid=(_N // _BN,),
        in_specs=[
            pl.BlockSpec((_NC, _BN, _F), lambda i: (0, i, 0)),
            pl.BlockSpec((_BN, 1), lambda i: (i, 0)),
            pl.BlockSpec((_BN, 1), lambda i: (i, 0)),
            pl.BlockSpec((_BN, _F), lambda i: (i, 0)),
            pl.BlockSpec((_BN, _F), lambda i: (i, 0)),
        ],
        out_specs=(
            pl.BlockSpec((_BN, _F), lambda i: (i, 0)),
            pl.BlockSpec((_BN, _F), lambda i: (i, 0)),
        ),
        out_shape=(
            jax.ShapeDtypeStruct((_N, _F), jnp.float32),
            jax.ShapeDtypeStruct((_N, _F), jnp.float32),
        ),
    )(s, d1, d2, tx_prev, g_prev)


_BM = 1000


def _matmul_call(x, tx1, tx2, tx3, s3, d1, W, b):
    def body(x_ref, t1_ref, t2_ref, t3_ref, s3_ref, d_ref, w_ref, b_ref, o_ref):
        tx4 = -2.0 * d_ref[...] * (s3_ref[0] + s3_ref[1]) - t2_ref[...]
        acc = jnp.dot(x_ref[...], w_ref[0], preferred_element_type=jnp.float32)
        acc += jnp.dot(t1_ref[...], w_ref[1], preferred_element_type=jnp.float32)
        acc += jnp.dot(t2_ref[...], w_ref[2], preferred_element_type=jnp.float32)
        acc += jnp.dot(t3_ref[...], w_ref[3], preferred_element_type=jnp.float32)
        acc += jnp.dot(tx4, w_ref[4], preferred_element_type=jnp.float32)
        o_ref[...] = acc + b_ref[...]

    return pl.pallas_call(
        body,
        grid=(_N // _BM,),
        in_specs=[
            pl.BlockSpec((_BM, _F), lambda i: (i, 0)),
            pl.BlockSpec((_BM, _F), lambda i: (i, 0)),
            pl.BlockSpec((_BM, _F), lambda i: (i, 0)),
            pl.BlockSpec((_BM, _F), lambda i: (i, 0)),
            pl.BlockSpec((_NC, _BM, _F), lambda i: (0, i, 0)),
            pl.BlockSpec((_BM, 1), lambda i: (i, 0)),
            pl.BlockSpec((_K, _F, _F), lambda i: (0, 0, 0)),
            pl.BlockSpec((1, _F), lambda i: (0, 0)),
        ],
        out_specs=pl.BlockSpec((_BM, _F), lambda i: (i, 0)),
        out_shape=jax.ShapeDtypeStruct((_N, _F), jnp.float32),
    )(x, tx1, tx2, tx3, s3, d1, W, b)


def kernel(x, adj, W, b):
    assert x.shape == (_N, _F) and adj.shape == (2, _E) and W.shape[0] == _K
    adj = adj.astype(jnp.int32)
    src_f = adj[0]
    dst_f = adj[1]

    degp = _deg_kernel(dst_f)
    d1_full, d2_full = _dinv_call(degp)
    d1 = d1_full[:_N, 0:1]
    d2 = d2_full[:_N, 0:1]

    g0 = _scale_call(x, d1)
    s0 = _prop(g0, src_f, dst_f)
    tx1, g1 = _step0_call(s0, d1, d2)
    s1 = _prop(g1, src_f, dst_f)
    tx2, g2 = _stepk_call(s1, d1, d2, x, g0)
    s2 = _prop(g2, src_f, dst_f)
    tx3, g3 = _stepk_call(s2, d1, d2, tx1, g1)
    s3 = _prop(g3, src_f, dst_f)
    out = _matmul_call(x, tx1, tx2, tx3, s3, d1, W, b.reshape(1, _F))
    return out


# speedup vs baseline: 15.8756x; 1.0148x over previous
import functools

import jax
import jax.numpy as jnp
from jax import lax
from jax.experimental import pallas as pl
from jax.experimental.pallas import tpu as pltpu
from jax.experimental.pallas import tpu_sc as plsc

_N = 10000
_E = 320000
_F = 128
_K = 5

_NC = 2
_NS = 16
_NW = _NC * _NS
_EPT = _E // _NW
_CH = 80
_NCHUNK = _EPT // _CH
_NPAD = 10240
_DW = 32

_mesh = plsc.VectorSubcoreMesh(core_axis_name="c", subcore_axis_name="s")


@functools.partial(
    pl.kernel,
    out_type=jax.ShapeDtypeStruct((_NC, _NPAD, _F), jnp.float32),
    mesh=_mesh,
    scratch_types=(
        [pltpu.VMEM((_CH,), jnp.int32) for _ in range(3)]
        + [pltpu.VMEM((_CH, _F), jnp.float32)]
        + [pltpu.VMEM_SHARED((_NPAD, _F), jnp.float32)]
        + [pltpu.SemaphoreType.DMA for _ in range(6)]
    ),
)
def _deg_kernel(dst_hbm, out_hbm, dch0, dch1, dch2, ones_v, acc_sp,
                sd0, sd1, sd2, sc0, sc1, sc2):
    cid = lax.axis_index("c")
    sid = lax.axis_index("s")
    wid = cid * _NS + sid
    rows = _NPAD // _NS
    ebase = wid * _EPT
    last = _NCHUNK - 1
    dch = [dch0, dch1, dch2]
    sd = [sd0, sd1, sd2]
    sc = [sc0, sc1, sc2]

    zeros16 = jnp.zeros((16,), jnp.float32)
    ones16 = jnp.ones((16,), jnp.float32)

    def zero_body(i, _):
        ones_v[i >> 3, pl.ds((i & 7) * 16, 16)] = zeros16
        return 0

    lax.fori_loop(0, _CH * (_F // 16), zero_body, 0)
    for t in range(rows // _CH):
        pltpu.sync_copy(ones_v, acc_sp.at[pl.ds(sid * rows + t * _CH, _CH)])

    def ones_body(i, _):
        ones_v[i >> 3, pl.ds((i & 7) * 16, 16)] = ones16
        return 0

    lax.fori_loop(0, _CH * (_F // 16), ones_body, 0)
    plsc.subcore_barrier()

    def didx(j, x):
        return pltpu.make_async_copy(
            dst_hbm.at[pl.ds(ebase + j * _CH, _CH)], dch[x], sd[x])

    def scat_start(x):
        pltpu.async_copy(ones_v, acc_sp.at[dch[x]], sc[x], add=True)

    def scat_wait(x):
        pltpu.make_async_copy(ones_v, acc_sp.at[dch[x]], sc[x]).wait()

    for x in range(3):
        didx(x, x).start()

    def body(t, _):
        j3 = 3 * t
        for x in range(3):
            didx(j3 + x, x).wait()
            scat_start(x)
        for x in range(3):
            scat_wait(x)
            didx(jnp.minimum(j3 + 3 + x, last), x).start()
        return 0

    nbody = (_NCHUNK - 2) // 3
    lax.fori_loop(0, nbody, body, 0)
    didx(last - 1, 0).wait()
    scat_start(0)
    didx(last, 1).wait()
    scat_start(1)
    didx(last, 2).wait()
    scat_wait(0)
    scat_wait(1)
    plsc.subcore_barrier()

    for t in range(rows // _CH):
        base = sid * rows + t * _CH
        pltpu.sync_copy(acc_sp.at[pl.ds(base, _CH)], out_hbm.at[cid, pl.ds(base, _CH)])


@functools.partial(
    pl.kernel,
    out_type=jax.ShapeDtypeStruct((_NC, _NPAD, _F), jnp.float32),
    mesh=_mesh,
    scratch_types=(
        [pltpu.VMEM((_CH,), jnp.int32) for _ in range(3)]
        + [pltpu.VMEM((_CH,), jnp.int32) for _ in range(3)]
        + [pltpu.VMEM((_CH, _F), jnp.float32) for _ in range(3)]
        + [pltpu.VMEM_SHARED((_NPAD, _F), jnp.float32)]
        + [pltpu.SemaphoreType.DMA for _ in range(12)]
    ),
)
def _prop_kernel(g_hbm, src_hbm, dst_hbm, out_hbm,
                 sch0, sch1, sch2, dch0, dch1, dch2, rw0, rw1, rw2, acc_sp,
                 ss0, ss1, ss2, sd0, sd1, sd2, sg0, sg1, sg2, sc0, sc1, sc2):
    cid = lax.axis_index("c")
    sid = lax.axis_index("s")
    wid = cid * _NS + sid
    rows = _NPAD // _NS
    ebase = wid * _EPT
    last = _NCHUNK - 1

    sch = [sch0, sch1, sch2]
    dch = [dch0, dch1, dch2]
    rw = [rw0, rw1, rw2]
    ss = [ss0, ss1, ss2]
    sd = [sd0, sd1, sd2]
    sg = [sg0, sg1, sg2]
    sc = [sc0, sc1, sc2]

    zeros16 = jnp.zeros((16,), jnp.float32)

    def zero_body(i, _):
        rw0[i >> 3, pl.ds((i & 7) * 16, 16)] = zeros16
        return 0

    lax.fori_loop(0, _CH * (_F // 16), zero_body, 0)
    for t in range(rows // _CH):
        pltpu.sync_copy(rw0, acc_sp.at[pl.ds(sid * rows + t * _CH, _CH)])
    plsc.subcore_barrier()

    def sidx(j, x):
        return pltpu.make_async_copy(
            src_hbm.at[pl.ds(ebase + j * _CH, _CH)], sch[x], ss[x])

    def didx(j, x):
        return pltpu.make_async_copy(
            dst_hbm.at[pl.ds(ebase + j * _CH, _CH)], dch[x], sd[x])

    def gath(x):
        return pltpu.make_async_copy(g_hbm.at[sch[x]], rw[x], sg[x])

    def scat_start(x):
        pltpu.async_copy(rw[x], acc_sp.at[dch[x]], sc[x], add=True)

    def scat_wait(x):
        pltpu.make_async_copy(rw[x], acc_sp.at[dch[x]], sc[x]).wait()

    for x in range(3):
        sidx(x, x).start()
        didx(x, x).start()
    for x in range(3):
        sidx(x, x).wait()
        gath(x).start()

    def body(t, _):
        j3 = 3 * t
        for x in range(3):
            gath(x).wait()
            sidx(jnp.minimum(j3 + 3 + x, last), x).start()
            didx(j3 + x, x).wait()
            scat_start(x)
        for x in range(3):
            scat_wait(x)
            didx(jnp.minimum(j3 + 3 + x, last), x).start()
            sidx(jnp.minimum(j3 + 3 + x, last), x).wait()
            gath(x).start()
        return 0

    nbody = (_NCHUNK - 2) // 3
    lax.fori_loop(0, nbody, body, 0)
    gath(0).wait()
    didx(last - 1, 0).wait()
    scat_start(0)
    gath(1).wait()
    didx(last, 1).wait()
    scat_start(1)
    gath(2).wait()
    didx(last, 2).wait()
    scat_wait(0)
    scat_wait(1)

    plsc.subcore_barrier()
    for t in range(rows // _CH):
        base = sid * rows + t * _CH
        pltpu.sync_copy(acc_sp.at[pl.ds(base, _CH)], out_hbm.at[cid, pl.ds(base, _CH)])


def _prop(g, src_f, dst_f):
    return _prop_kernel(g, src_f, dst_f)


def _dinv_scale_call(degp, xpad):
    def body(p_ref, x_ref, d1_ref, d2_ref, g0_ref):
        deg = p_ref[0] + p_ref[1]
        d = jnp.where(deg > 0, lax.rsqrt(jnp.maximum(deg, 1.0)), 0.0)
        d1_ref[...] = d
        d2_ref[...] = d * d
        g0_ref[...] = x_ref[...] * d

    return pl.pallas_call(
        body,
        grid=(_NPAD // 2048,),
        in_specs=[
            pl.BlockSpec((_NC, 2048, _F), lambda i: (0, i, 0)),
            pl.BlockSpec((2048, _F), lambda i: (i, 0)),
        ],
        out_specs=(
            pl.BlockSpec((2048, _F), lambda i: (i, 0)),
            pl.BlockSpec((2048, _F), lambda i: (i, 0)),
            pl.BlockSpec((2048, _F), lambda i: (i, 0)),
        ),
        out_shape=(
            jax.ShapeDtypeStruct((_NPAD, _F), jnp.float32),
            jax.ShapeDtypeStruct((_NPAD, _F), jnp.float32),
            jax.ShapeDtypeStruct((_NPAD, _F), jnp.float32),
        ),
    )(degp, xpad)


_BN = 2000


def _gstep_call(s, d2, g_prev, first):
    def body(s_ref, d2_ref, gp_ref, g_ref):
        ssum = s_ref[0] + s_ref[1]
        if first:
            g_ref[...] = -d2_ref[...] * ssum
        else:
            g_ref[...] = -2.0 * d2_ref[...] * ssum - gp_ref[...]

    return pl.pallas_call(
        body,
        grid=(_N // _BN,),
        in_specs=[
            pl.BlockSpec((_NC, _BN, _F), lambda i: (0, i, 0)),
            pl.BlockSpec((_BN, 1), lambda i: (i, 0)),
            pl.BlockSpec((_BN, _F), lambda i: (i, 0)),
        ],
        out_specs=pl.BlockSpec((_BN, _F), lambda i: (i, 0)),
        out_shape=jax.ShapeDtypeStruct((_N, _F), jnp.float32),
    )(s, d2, g_prev)


def _txstep_call(s, d1, tx_prev, first):
    def body(s_ref, d1_ref, tp_ref, tx_ref):
        ssum = s_ref[0] + s_ref[1]
        if first:
            tx_ref[...] = -d1_ref[...] * ssum
        else:
            tx_ref[...] = -2.0 * d1_ref[...] * ssum - tp_ref[...]

    return pl.pallas_call(
        body,
        grid=(_N // _BN,),
        in_specs=[
            pl.BlockSpec((_NC, _BN, _F), lambda i: (0, i, 0)),
            pl.BlockSpec((_BN, 1), lambda i: (i, 0)),
            pl.BlockSpec((_BN, _F), lambda i: (i, 0)),
        ],
        out_specs=pl.BlockSpec((_BN, _F), lambda i: (i, 0)),
        out_shape=jax.ShapeDtypeStruct((_N, _F), jnp.float32),
    )(s, d1, tx_prev)


_BM = 1000


def _matmul_call(x, tx1, tx2, tx3, s3, d1, W, b):
    def body(x_ref, t1_ref, t2_ref, t3_ref, s3_ref, d_ref, w_ref, b_ref, o_ref):
        tx4 = -2.0 * d_ref[...] * (s3_ref[0] + s3_ref[1]) - t2_ref[...]
        acc = jnp.dot(x_ref[...], w_ref[0], preferred_element_type=jnp.float32)
        acc += jnp.dot(t1_ref[...], w_ref[1], preferred_element_type=jnp.float32)
        acc += jnp.dot(t2_ref[...], w_ref[2], preferred_element_type=jnp.float32)
        acc += jnp.dot(t3_ref[...], w_ref[3], preferred_element_type=jnp.float32)
        acc += jnp.dot(tx4, w_ref[4], preferred_element_type=jnp.float32)
        o_ref[...] = acc + b_ref[...]

    return pl.pallas_call(
        body,
        grid=(_N // _BM,),
        in_specs=[
            pl.BlockSpec((_BM, _F), lambda i: (i, 0)),
            pl.BlockSpec((_BM, _F), lambda i: (i, 0)),
            pl.BlockSpec((_BM, _F), lambda i: (i, 0)),
            pl.BlockSpec((_BM, _F), lambda i: (i, 0)),
            pl.BlockSpec((_NC, _BM, _F), lambda i: (0, i, 0)),
            pl.BlockSpec((_BM, 1), lambda i: (i, 0)),
            pl.BlockSpec((_K, _F, _F), lambda i: (0, 0, 0)),
            pl.BlockSpec((1, _F), lambda i: (0, 0)),
        ],
        out_specs=pl.BlockSpec((_BM, _F), lambda i: (i, 0)),
        out_shape=jax.ShapeDtypeStruct((_N, _F), jnp.float32),
    )(x, tx1, tx2, tx3, s3, d1, W, b)


def kernel(x, adj, W, b):
    assert x.shape == (_N, _F) and adj.shape == (2, _E) and W.shape[0] == _K
    adj = adj.astype(jnp.int32)
    src_f = adj[0]
    dst_f = adj[1]
    xpad = jnp.pad(x, ((0, _NPAD - _N), (0, 0)))

    degp = _deg_kernel(dst_f)
    d1f, d2f, g0p = _dinv_scale_call(degp, xpad)
    d1 = d1f[:_N, 0:1]
    d2 = d2f[:_N, 0:1]

    s0 = _prop(g0p, src_f, dst_f)
    g1 = _gstep_call(s0, d2, x, first=True)
    tx1 = _txstep_call(s0, d1, x, first=True)
    s1 = _prop(g1, src_f, dst_f)
    g2 = _gstep_call(s1, d2, g0p[:_N], first=False)
    tx2 = _txstep_call(s1, d1, x, first=False)
    s2 = _prop(g2, src_f, dst_f)
    g3 = _gstep_call(s2, d2, g1, first=False)
    tx3 = _txstep_call(s2, d1, tx1, first=False)
    s3 = _prop(g3, src_f, dst_f)
    out = _matmul_call(x, tx1, tx2, tx3, s3, d1, W, b.reshape(1, _F))
    return out
